# baseline (speedup 1.0000x reference)
import functools

import jax
import jax.numpy as jnp
from jax import lax
from jax.experimental import pallas as pl
from jax.experimental.pallas import tpu as pltpu
from jax.experimental.pallas import tpu_sc as plsc

_NC = 2
_NS = 16
_NW = _NC * _NS
_L = 16


@functools.lru_cache(maxsize=None)
def _build_sc_call(B, N, M):
    E = B * M
    NT = B * N
    W = 2 * E
    WPW = W // _NW
    EPW = E // _NW
    GW = 2 * M
    assert W % _NW == 0 and WPW % _L == 0 and WPW % 8 == 0
    assert E % _NW == 0 and EPW % _L == 0 and EPW % 8 == 0
    assert GW % _L == 0 and M % _L == 0
    NPC = 320
    NREM = NT - NPC * (_NW - 1)
    assert 0 < NREM <= NPC and NPC % _L == 0 and NREM % 8 == 0
    NVEC = NPC // _L
    LENB = ((B + _L - 1) // _L) * _L

    mesh = plsc.VectorSubcoreMesh(core_axis_name="c", subcore_axis_name="s")

    @functools.partial(
        pl.kernel,
        mesh=mesh,
        compiler_params=pltpu.CompilerParams(
            needs_layout_passes=False, use_tc_tiling_on_sc=False),
        out_type=[
            jax.ShapeDtypeStruct((W,), jnp.int32),
            jax.ShapeDtypeStruct((E,), jnp.int32),
            jax.ShapeDtypeStruct((E,), jnp.int32),
            jax.ShapeDtypeStruct((NT,), jnp.int32),
            jax.ShapeDtypeStruct((NT,), jnp.int32),
            jax.ShapeDtypeStruct((B,), jnp.int32),
            jax.ShapeDtypeStruct((B,), jnp.int32),
        ],
        scratch_types=[
            pltpu.VMEM((WPW,), jnp.int32),
            pltpu.VMEM((WPW,), jnp.int32),
            pltpu.VMEM((EPW,), jnp.int32),
            pltpu.VMEM((EPW,), jnp.int32),
            pltpu.VMEM((NPC,), jnp.int32),
            pltpu.VMEM((NPC,), jnp.int32),
            pltpu.VMEM((LENB,), jnp.int32),
            pltpu.SemaphoreType.DMA,
            pltpu.SemaphoreType.DMA,
            pltpu.SemaphoreType.DMA,
        ],
    )
    def sc_fn(y_hbm, z_hbm, gie_hbm, eid_hbm, gin_hbm, nid_hbm,
              nl_hbm, el_hbm, inb, zb, gieb, eidb, gnb, nnb, lenb,
              sem_in, sem_small, sem_out):
        wid = lax.axis_index("s") * _NC + lax.axis_index("c")
        iota = lax.iota(jnp.int32, _L)

        wbase = wid * WPW
        cin = pltpu.async_copy(y_hbm.at[pl.ds(wbase, WPW)], inb, sem_in)

        ebase = wid * EPW

        IDU = 5
        assert EPW % (_L * IDU) == 0

        def id_body(j, _):
            for u in range(IDU):
                p = (j * IDU + u) * _L
                e0 = ebase + p
                g = e0 // M
                gieb[pl.ds(p, _L)] = jnp.broadcast_to(g, (_L,))
                eidb[pl.ds(p, _L)] = (e0 - g * M) + iota
            return 0

        lax.fori_loop(0, EPW // (_L * IDU), id_body, 0)
        cid0 = pltpu.async_copy(gieb, gie_hbm.at[pl.ds(ebase, EPW)], sem_out)
        cid1 = pltpu.async_copy(eidb, eid_hbm.at[pl.ds(ebase, EPW)], sem_out)

        nbase = wid * NPC

        def node_body(j, _):
            v = (nbase + j * _L) + iota
            gg = v // N
            gnb[pl.ds(j * _L, _L)] = gg
            nnb[pl.ds(j * _L, _L)] = v - gg * N
            return 0

        lax.fori_loop(0, NVEC, node_body, 0)

        @pl.when(wid < _NW - 1)
        def _node_full():
            c0 = pltpu.async_copy(gnb, gin_hbm.at[pl.ds(nbase, NPC)],
                                  sem_small)
            c1 = pltpu.async_copy(nnb, nid_hbm.at[pl.ds(nbase, NPC)],
                                  sem_small)
            c0.wait()
            c1.wait()

        @pl.when(wid == _NW - 1)
        def _node_rem():
            c0 = pltpu.async_copy(gnb.at[pl.ds(0, NREM)],
                                  gin_hbm.at[pl.ds(nbase, NREM)], sem_small)
            c1 = pltpu.async_copy(nnb.at[pl.ds(0, NREM)],
                                  nid_hbm.at[pl.ds(nbase, NREM)], sem_small)
            c0.wait()
            c1.wait()

        @pl.when(wid == 0)
        def _node_len():
            for j in range(LENB // _L):
                lenb[pl.ds(j * _L, _L)] = jnp.full((_L,), N, jnp.int32)
            pltpu.async_copy(lenb.at[pl.ds(0, B)], nl_hbm, sem_small).wait()

        @pl.when(wid == 1)
        def _edge_len():
            for j in range(LENB // _L):
                lenb[pl.ds(j * _L, _L)] = jnp.full((_L,), M, jnp.int32)
            pltpu.async_copy(lenb.at[pl.ds(0, B)], el_hbm, sem_small).wait()

        cin.wait()

        ADU = 5
        assert WPW % (_L * ADU) == 0

        def add_body(j, _):
            for u in range(ADU):
                p = (j * ADU + u) * _L
                off = ((wbase + p) // GW) * N
                zb[pl.ds(p, _L)] = inb[pl.ds(p, _L)] + off
            return 0

        lax.fori_loop(0, WPW // (_L * ADU), add_body, 0)

        cz = pltpu.async_copy(zb, z_hbm.at[pl.ds(wbase, WPW)], sem_out)
        cid0.wait()
        cid1.wait()
        cz.wait()

    return sc_fn


def kernel(nodes, edge_indices):
    B, N, F = nodes.shape
    _, M, _ = edge_indices.shape
    E = B * M
    idt = edge_indices.dtype

    nodes_flatten = nodes.reshape(B * N, F)

    y = (edge_indices.astype(jnp.int32)
         .reshape(B, M // 128, 128, 2)
         .transpose(0, 1, 3, 2)
         .reshape(-1))

    sc_fn = _build_sc_call(B, N, M)
    z, gie, eid, gin, nid, nl, el = sc_fn(y)

    disjoint_indices = (z.reshape(B * M // 128, 2, 128)
                        .transpose(1, 0, 2)
                        .reshape(2, E)
                        .astype(idt))
    return (nodes_flatten, disjoint_indices, gin, gie, nid, eid, nl, el)
